# Initial kernel scaffold; baseline (speedup 1.0000x reference)
import jax
import jax.numpy as jnp
from jax.experimental import pallas as pl


def kernel(x, edge_index, batch, bn1_g, bn1_b, bn1_m, bn1_v, W1, b1, W2, b2, bn2_g, bn2_b, bn2_m, bn2_v, W3, b3, bn3_g, bn3_b, bn3_m, bn3_v, W4, b4, bn4_g, bn4_b, bn4_m, bn4_v):
    raise NotImplementedError("write your pallas kernel here")


# speedup vs baseline: 1.6659x; 1.6659x over previous
import functools

import jax
import jax.numpy as jnp
from jax.experimental import pallas as pl
from jax.experimental.pallas import tpu as pltpu

N = 100000
D_IN = 56
NUM_GRAPHS = 512
ROWS = 2000
STEPS = N // ROWS


def _fused_kernel(x_ref, batch_ref,
                  bn1_g, bn1_b, bn1_m, bn1_v, W1, b1,
                  W2, b2, bn2_g, bn2_b, bn2_m, bn2_v,
                  W3, b3, bn3_g, bn3_b, bn3_m, bn3_v,
                  W4, b4, bn4_g, bn4_b, bn4_m, bn4_v,
                  out_ref, acc_ref, cnt_ref):
    i = pl.program_id(0)

    @pl.when(i == 0)
    def _init():
        acc_ref[...] = jnp.zeros_like(acc_ref)
        cnt_ref[...] = jnp.zeros_like(cnt_ref)

    a = bn1_g[...] * jax.lax.rsqrt(bn1_v[...] + 1e-5)
    c = bn1_b[...] - bn1_m[...] * a
    h = x_ref[...] * a + c
    h = jnp.where(h > 0, h, jnp.exp(h) - 1.0)

    seg = batch_ref[0]
    gid = jax.lax.broadcasted_iota(jnp.int32, (NUM_GRAPHS, ROWS), 0)
    onehot = (seg == gid).astype(jnp.float32)
    acc_ref[...] += jnp.dot(onehot, h, preferred_element_type=jnp.float32, precision=jax.lax.Precision.HIGHEST)
    cnt_ref[...] += jnp.sum(onehot, axis=1, keepdims=True)

    @pl.when(i == STEPS - 1)
    def _epilogue():
        pooled = jnp.dot(acc_ref[...], W1[...],
                         preferred_element_type=jnp.float32, precision=jax.lax.Precision.HIGHEST)
        pooled += cnt_ref[...] * b1[...]
        z = jnp.dot(pooled, W2[...], preferred_element_type=jnp.float32, precision=jax.lax.Precision.HIGHEST)
        z += b2[...]
        a2 = bn2_g[...] * jax.lax.rsqrt(bn2_v[...] + 1e-5)
        z = jnp.maximum(z * a2 + (bn2_b[...] - bn2_m[...] * a2), 0.0)
        z = jnp.dot(z, W3[...], preferred_element_type=jnp.float32, precision=jax.lax.Precision.HIGHEST)
        z += b3[...]
        a3 = bn3_g[...] * jax.lax.rsqrt(bn3_v[...] + 1e-5)
        z = jnp.maximum(z * a3 + (bn3_b[...] - bn3_m[...] * a3), 0.0)
        z = jnp.dot(z, W4[...], preferred_element_type=jnp.float32, precision=jax.lax.Precision.HIGHEST)
        z += b4[...]
        a4 = bn4_g[...] * jax.lax.rsqrt(bn4_v[...] + 1e-5)
        out_ref[...] = z * a4 + (bn4_b[...] - bn4_m[...] * a4)


def kernel(x, edge_index, batch,
           bn1_g, bn1_b, bn1_m, bn1_v, W1, b1,
           W2, b2, bn2_g, bn2_b, bn2_m, bn2_v,
           W3, b3, bn3_g, bn3_b, bn3_m, bn3_v,
           W4, b4, bn4_g, bn4_b, bn4_m, bn4_v):
    del edge_index
    batch3 = batch.reshape(STEPS, 1, ROWS)
    row = lambda v: v.reshape(1, -1)

    full = lambda shape: pl.BlockSpec(shape, lambda i: (0,) * len(shape))
    out = pl.pallas_call(
        _fused_kernel,
        grid=(STEPS,),
        in_specs=[
            pl.BlockSpec((ROWS, D_IN), lambda i: (i, 0)),
            pl.BlockSpec((1, 1, ROWS), lambda i: (i, 0, 0)),
            full((1, D_IN)), full((1, D_IN)), full((1, D_IN)), full((1, D_IN)),
            full(W1.shape), full((1, 64)),
            full(W2.shape), full((1, 128)),
            full((1, 128)), full((1, 128)), full((1, 128)), full((1, 128)),
            full(W3.shape), full((1, 64)),
            full((1, 64)), full((1, 64)), full((1, 64)), full((1, 64)),
            full(W4.shape), full((1, 1)),
            full((1, 1)), full((1, 1)), full((1, 1)), full((1, 1)),
        ],
        out_specs=pl.BlockSpec((NUM_GRAPHS, 1), lambda i: (0, 0)),
        out_shape=jax.ShapeDtypeStruct((NUM_GRAPHS, 1), jnp.float32),
        scratch_shapes=[
            pltpu.VMEM((NUM_GRAPHS, D_IN), jnp.float32),
            pltpu.VMEM((NUM_GRAPHS, 1), jnp.float32),
        ],
    )(x, batch3,
      row(bn1_g), row(bn1_b), row(bn1_m), row(bn1_v), W1, row(b1),
      W2, row(b2), row(bn2_g), row(bn2_b), row(bn2_m), row(bn2_v),
      W3, row(b3), row(bn3_g), row(bn3_b), row(bn3_m), row(bn3_v),
      W4, row(b4), row(bn4_g), row(bn4_b), row(bn4_m), row(bn4_v))
    return out.reshape(-1)


# speedup vs baseline: 2.5874x; 1.5532x over previous
import jax
import jax.numpy as jnp
from jax.experimental import pallas as pl
from jax.experimental.pallas import tpu as pltpu

N = 100000
NP = 102400
D_IN = 56
DP = 64
NUM_GRAPHS = 512
ROWS = 2048
STEPS = NP // ROWS
HI = jax.lax.Precision.HIGHEST


def _fused_kernel(xT_ref, batch_ref, a_ref, c_ref, W1a,
                  W2, b2, bn2_g, bn2_b, bn2_m, bn2_v,
                  W3, b3, bn3_g, bn3_b, bn3_m, bn3_v,
                  W4, b4, bn4_g, bn4_b, bn4_m, bn4_v,
                  out_ref, acc_ref):
    i = pl.program_id(0)

    @pl.when(i == 0)
    def _init():
        acc_ref[...] = jnp.zeros_like(acc_ref)

    h = xT_ref[...] * a_ref[...] + c_ref[...]
    h = jnp.where(h > 0, h, jnp.exp(h) - 1.0)
    h_hi = h.astype(jnp.bfloat16)
    h_lo = (h - h_hi.astype(jnp.float32)).astype(jnp.bfloat16)

    seg = batch_ref[0]
    gid = jax.lax.broadcasted_iota(jnp.int16, (ROWS, NUM_GRAPHS), 1)
    onehot = jnp.where(seg == gid, jnp.bfloat16(1), jnp.bfloat16(0))
    acc_ref[...] += (
        jnp.dot(h_hi, onehot, preferred_element_type=jnp.float32)
        + jnp.dot(h_lo, onehot, preferred_element_type=jnp.float32))

    @pl.when(i == STEPS - 1)
    def _epilogue():
        pooled = jax.lax.dot_general(
            acc_ref[...], W1a[...], (((0,), (0,)), ((), ())),
            precision=HI, preferred_element_type=jnp.float32)
        z = jnp.dot(pooled, W2[...], precision=HI,
                    preferred_element_type=jnp.float32)
        z += b2[...]
        a2 = bn2_g[...] * jax.lax.rsqrt(bn2_v[...] + 1e-5)
        z = jnp.maximum(z * a2 + (bn2_b[...] - bn2_m[...] * a2), 0.0)
        z = jnp.dot(z, W3[...], precision=HI,
                    preferred_element_type=jnp.float32)
        z += b3[...]
        a3 = bn3_g[...] * jax.lax.rsqrt(bn3_v[...] + 1e-5)
        z = jnp.maximum(z * a3 + (bn3_b[...] - bn3_m[...] * a3), 0.0)
        z = jnp.dot(z, W4[...], precision=HI,
                    preferred_element_type=jnp.float32)
        z += b4[...]
        a4 = bn4_g[...] * jax.lax.rsqrt(bn4_v[...] + 1e-5)
        out_ref[...] = z * a4 + (bn4_b[...] - bn4_m[...] * a4)


def kernel(x, edge_index, batch,
           bn1_g, bn1_b, bn1_m, bn1_v, W1, b1,
           W2, b2, bn2_g, bn2_b, bn2_m, bn2_v,
           W3, b3, bn3_g, bn3_b, bn3_m, bn3_v,
           W4, b4, bn4_g, bn4_b, bn4_m, bn4_v):
    del edge_index
    xT = jnp.pad(x.T, ((0, DP - D_IN), (0, NP - N)))
    batch3 = jnp.pad(batch.astype(jnp.int16), (0, NP - N),
                     constant_values=jnp.int16(NUM_GRAPHS)
                     ).reshape(STEPS, ROWS, 1)
    a = bn1_g * jax.lax.rsqrt(bn1_v + 1e-5)
    c = bn1_b - bn1_m * a
    a_pad = jnp.pad(a, (0, DP - D_IN)).reshape(DP, 1)
    c_pad = jnp.pad(c, (0, DP - D_IN)).at[D_IN].set(1.0).reshape(DP, 1)
    W1a = jnp.concatenate(
        [W1, b1[None, :], jnp.zeros((DP - D_IN - 1, 64), jnp.float32)], axis=0)
    row = lambda v: v.reshape(1, -1)

    full = lambda shape: pl.BlockSpec(shape, lambda i: (0,) * len(shape))
    out = pl.pallas_call(
        _fused_kernel,
        grid=(STEPS,),
        in_specs=[
            pl.BlockSpec((DP, ROWS), lambda i: (0, i)),
            pl.BlockSpec((1, ROWS, 1), lambda i: (i, 0, 0)),
            full((DP, 1)), full((DP, 1)), full((DP, 64)),
            full(W2.shape), full((1, 128)),
            full((1, 128)), full((1, 128)), full((1, 128)), full((1, 128)),
            full(W3.shape), full((1, 64)),
            full((1, 64)), full((1, 64)), full((1, 64)), full((1, 64)),
            full(W4.shape), full((1, 1)),
            full((1, 1)), full((1, 1)), full((1, 1)), full((1, 1)),
        ],
        out_specs=pl.BlockSpec((NUM_GRAPHS, 1), lambda i: (0, 0)),
        out_shape=jax.ShapeDtypeStruct((NUM_GRAPHS, 1), jnp.float32),
        scratch_shapes=[
            pltpu.VMEM((DP, NUM_GRAPHS), jnp.float32),
        ],
    )(xT, batch3, a_pad, c_pad, W1a,
      W2, row(b2), row(bn2_g), row(bn2_b), row(bn2_m), row(bn2_v),
      W3, row(b3), row(bn3_g), row(bn3_b), row(bn3_m), row(bn3_v),
      W4, row(b4), row(bn4_g), row(bn4_b), row(bn4_m), row(bn4_v))
    return out.reshape(-1)
